# baseline (speedup 1.0000x reference)
import functools

import jax
import jax.numpy as jnp
from jax import lax
from jax.experimental import pallas as pl
from jax.experimental.pallas import tpu as pltpu
from jax.experimental.pallas import tpu_sc as plsc

_N = 10000
_E = 320000
_D = 128
_NC = 2
_NS = 16
_NW = _NC * _NS
_CHUNK = 128
_NCHUNKS = _E // _CHUNK
_CPW = -(-_NCHUNKS // _NW)
_NP = 10240
_BLK = 8
_PCHUNKS = 2560
_EPAD = _PCHUNKS * _CHUNK
_BPW = _PCHUNKS // (_NW * _BLK)
_RPT = _NP // _NS
_DEGW = 16


def _sc_mesh():
    return plsc.VectorSubcoreMesh(
        core_axis_name="c", subcore_axis_name="s",
        num_cores=_NC, num_subcores=_NS)


_DCP = -(-_NCHUNKS // _NS)


def _degree_partials(src, dst, zeros_nd, ones_nd):
    @functools.partial(
        pl.kernel,
        out_type=jax.ShapeDtypeStruct((_NC, _NP, _D), jnp.float32),
        mesh=_sc_mesh(),
        scratch_types=[
            pltpu.VMEM((_BLK, _CHUNK), jnp.int32),
            pltpu.VMEM((_CHUNK, _D), jnp.float32),
            pltpu.VMEM_SHARED((_NP, _D), jnp.float32),
        ],
    )
    def deg_k(src_hbm, dst_hbm, z_hbm, o_hbm, out_hbm, idx, ones_v, acc_sh):
        cid = lax.axis_index("c")
        sid = lax.axis_index("s")
        r0 = sid * _RPT
        pltpu.sync_copy(z_hbm.at[pl.ds(r0, _RPT)], acc_sh.at[pl.ds(r0, _RPT)])
        pltpu.sync_copy(o_hbm, ones_v)
        plsc.subcore_barrier()
        nblk = _PCHUNKS // (_NS * _BLK)

        def body(j, carry):
            b = j * _NS + sid

            @pl.when(cid == 0)
            def _():
                pltpu.sync_copy(src_hbm.at[pl.ds(b * _BLK, _BLK)], idx)

            @pl.when(cid == 1)
            def _():
                pltpu.sync_copy(dst_hbm.at[pl.ds(b * _BLK, _BLK)], idx)

            for k in range(_BLK):
                pltpu.sync_copy(ones_v, acc_sh.at[idx.at[k]], add=True)
            return carry

        lax.fori_loop(0, nblk, body, 0)
        plsc.subcore_barrier()
        pltpu.sync_copy(acc_sh.at[pl.ds(r0, _RPT)],
                        out_hbm.at[cid, pl.ds(r0, _RPT)])

    return deg_k(src, dst, zeros_nd, ones_nd)


def _aggregate_partials(xs, src2d, dst2d, zeros_nd):
    @functools.partial(
        pl.kernel,
        out_type=jax.ShapeDtypeStruct((_NC, _NP, _D), jnp.float32),
        mesh=_sc_mesh(),
        scratch_types=[
            pltpu.VMEM((_BLK, _CHUNK), jnp.int32),
            pltpu.VMEM((_BLK, _CHUNK), jnp.int32),
            pltpu.VMEM((_CHUNK, _D), jnp.float32),
            pltpu.VMEM((_CHUNK, _D), jnp.float32),
            pltpu.VMEM_SHARED((_NP, _D), jnp.float32),
            pltpu.SemaphoreType.DMA,
            pltpu.SemaphoreType.DMA,
        ],
    )
    def agg_k(xs_hbm, src_hbm, dst_hbm, z_hbm, out_hbm,
              sidx, didx, rows0, rows1, agg_sh, sem0, sem1):
        cid = lax.axis_index("c")
        sid = lax.axis_index("s")
        wid = cid * _NS + sid
        r0 = sid * _RPT
        pltpu.sync_copy(z_hbm.at[pl.ds(r0, _RPT)], agg_sh.at[pl.ds(r0, _RPT)])
        plsc.subcore_barrier()
        bufs = (rows0, rows1)
        sems = (sem0, sem1)

        def body(j, carry):
            b = j * _NW + wid
            pltpu.sync_copy(src_hbm.at[pl.ds(b * _BLK, _BLK)], sidx)
            pltpu.sync_copy(dst_hbm.at[pl.ds(b * _BLK, _BLK)], didx)
            descs = [None] * _BLK
            descs[0] = pltpu.async_copy(xs_hbm.at[sidx.at[0]], bufs[0], sems[0])
            for k in range(_BLK):
                if k + 1 < _BLK:
                    descs[k + 1] = pltpu.async_copy(
                        xs_hbm.at[sidx.at[k + 1]], bufs[(k + 1) % 2],
                        sems[(k + 1) % 2])
                descs[k].wait()
                pltpu.sync_copy(bufs[k % 2], agg_sh.at[didx.at[k]], add=True)
            return carry

        lax.fori_loop(0, _BPW, body, 0)
        plsc.subcore_barrier()
        pltpu.sync_copy(agg_sh.at[pl.ds(r0, _RPT)],
                        out_hbm.at[cid, pl.ds(r0, _RPT)])

    return agg_k(xs, src2d, dst2d, zeros_nd)


def _emb_body(h_ref, w_ref, b_ref, degp_ref, x_ref, xs_ref, cin_ref, cout_ref):
    x = jnp.dot(h_ref[...], w_ref[...],
                preferred_element_type=jnp.float32) + b_ref[...]
    deg_out = degp_ref[0, 0:_N, 0:1]
    deg_in = degp_ref[1, 0:_N, 0:1]
    c_out = lax.rsqrt(jnp.maximum(deg_out, 1.0))
    c_in = lax.rsqrt(jnp.maximum(deg_in, 1.0))
    x_ref[...] = x
    xs_ref[0:_N] = x * c_out
    xs_ref[_N:_NP] = jnp.zeros((_NP - _N, _D), jnp.float32)
    cin_ref[...] = c_in
    cout_ref[...] = c_out


def _embed(h, W_emb, b_emb, degp):
    return pl.pallas_call(
        _emb_body,
        out_shape=[
            jax.ShapeDtypeStruct((_N, _D), jnp.float32),
            jax.ShapeDtypeStruct((_NP, _D), jnp.float32),
            jax.ShapeDtypeStruct((_N, 1), jnp.float32),
            jax.ShapeDtypeStruct((_N, 1), jnp.float32),
        ],
    )(h, W_emb, b_emb.reshape(1, _D), degp)


def _layer_body(p_ref, cin_ref, cout_ref, w_ref, b_ref, g_ref, bt_ref,
                xin_ref, xout_ref, xsout_ref):
    agg = (p_ref[0, 0:_N] + p_ref[1, 0:_N]) * cin_ref[...]
    y = jnp.dot(agg, w_ref[...],
                preferred_element_type=jnp.float32) + b_ref[...]
    mu = jnp.mean(y, axis=0, keepdims=True)
    yc = y - mu
    var = jnp.mean(yc * yc, axis=0, keepdims=True)
    yn = yc * lax.rsqrt(var + 1e-5) * g_ref[...] + bt_ref[...]
    x_new = xin_ref[...] + jnp.maximum(yn, 0.0)
    xout_ref[...] = x_new
    xsout_ref[0:_N] = x_new * cout_ref[...]
    xsout_ref[_N:_NP] = jnp.zeros((_NP - _N, _D), jnp.float32)


def _layer(parts, c_in, c_out, W, b, g, bt, x_in):
    return pl.pallas_call(
        _layer_body,
        out_shape=[
            jax.ShapeDtypeStruct((_N, _D), jnp.float32),
            jax.ShapeDtypeStruct((_NP, _D), jnp.float32),
        ],
    )(parts, c_in, c_out, W, b.reshape(1, _D), g.reshape(1, _D),
      bt.reshape(1, _D), x_in)


def kernel(h, edge_index, e, W_emb, b_emb, Ws, bs, gammas, betas):
    del e
    pad = jnp.full((_EPAD - _E,), _NP - 1, jnp.int32)
    src2d = jnp.concatenate([edge_index[0], pad]).reshape(_PCHUNKS, _CHUNK)
    dst2d = jnp.concatenate([edge_index[1], pad]).reshape(_PCHUNKS, _CHUNK)
    zeros_nd = jnp.zeros((_NP, _D), jnp.float32)
    ones_nd = jnp.ones((_CHUNK, _D), jnp.float32)

    degp = _degree_partials(src2d, dst2d, zeros_nd, ones_nd)
    x, xs, c_in, c_out = _embed(h, W_emb, b_emb, degp)
    for l in range(4):
        parts = _aggregate_partials(xs, src2d, dst2d, zeros_nd)
        x, xs = _layer(parts, c_in, c_out, Ws[l], bs[l], gammas[l],
                       betas[l], x)
    return x


# speedup vs baseline: 2.4592x; 2.4592x over previous
import functools

import jax
import jax.numpy as jnp
from jax import lax
from jax.experimental import pallas as pl
from jax.experimental.pallas import tpu as pltpu
from jax.experimental.pallas import tpu_sc as plsc

_N = 10000
_E = 320000
_D = 128
_NC = 2
_NS = 16
_NW = _NC * _NS
_CHUNK = 128
_NCHUNKS = _E // _CHUNK
_CPW = -(-_NCHUNKS // _NW)
_NP = 10240
_BLK = 8
_PCHUNKS = 2560
_EPAD = _PCHUNKS * _CHUNK
_BPW = _PCHUNKS // (_NW * _BLK)
_RPT = _NP // _NS
_DEGW = 16


def _sc_mesh():
    return plsc.VectorSubcoreMesh(
        core_axis_name="c", subcore_axis_name="s",
        num_cores=_NC, num_subcores=_NS)


_DCP = -(-_NCHUNKS // _NS)


def _degree_partials(src, dst, zeros_nd, ones_nd):
    @functools.partial(
        pl.kernel,
        out_type=jax.ShapeDtypeStruct((_NC, _NP, _D), jnp.float32),
        mesh=_sc_mesh(),
        scratch_types=[
            pltpu.VMEM((_BLK, _CHUNK), jnp.int32),
            pltpu.VMEM((_CHUNK, _D), jnp.float32),
            pltpu.VMEM_SHARED((_NP, _D), jnp.float32),
        ],
    )
    def deg_k(src_hbm, dst_hbm, z_hbm, o_hbm, out_hbm, idx, ones_v, acc_sh):
        cid = lax.axis_index("c")
        sid = lax.axis_index("s")
        r0 = sid * _RPT
        pltpu.sync_copy(z_hbm.at[pl.ds(r0, _RPT)], acc_sh.at[pl.ds(r0, _RPT)])
        pltpu.sync_copy(o_hbm, ones_v)
        plsc.subcore_barrier()
        nblk = _PCHUNKS // (_NS * _BLK)

        def body(j, carry):
            b = j * _NS + sid

            @pl.when(cid == 0)
            def _():
                pltpu.sync_copy(src_hbm.at[pl.ds(b * _BLK, _BLK)], idx)

            @pl.when(cid == 1)
            def _():
                pltpu.sync_copy(dst_hbm.at[pl.ds(b * _BLK, _BLK)], idx)

            for k in range(_BLK):
                pltpu.sync_copy(ones_v, acc_sh.at[idx.at[k]], add=True)
            return carry

        lax.fori_loop(0, nblk, body, 0)
        plsc.subcore_barrier()
        pltpu.sync_copy(acc_sh.at[pl.ds(r0, _RPT)],
                        out_hbm.at[cid, pl.ds(r0, _RPT)])

    return deg_k(src, dst, zeros_nd, ones_nd)


def _aggregate_partials(xs, src2d, dst2d, zeros_nd):
    @functools.partial(
        pl.kernel,
        out_type=jax.ShapeDtypeStruct((_NC, _NP, _D), jnp.float32),
        mesh=_sc_mesh(),
        scratch_types=[
            pltpu.VMEM((_BLK, _CHUNK), jnp.int32),
            pltpu.VMEM((_BLK, _CHUNK), jnp.int32),
            pltpu.VMEM((_CHUNK, _D), jnp.float32),
            pltpu.VMEM((_CHUNK, _D), jnp.float32),
            pltpu.VMEM_SHARED((_NP, _D), jnp.float32),
            pltpu.SemaphoreType.DMA,
            pltpu.SemaphoreType.DMA,
        ],
    )
    def agg_k(xs_hbm, src_hbm, dst_hbm, z_hbm, out_hbm,
              sidx, didx, rows0, rows1, agg_sh, sem0, sem1):
        cid = lax.axis_index("c")
        sid = lax.axis_index("s")
        wid = cid * _NS + sid
        r0 = sid * _RPT
        pltpu.sync_copy(z_hbm.at[pl.ds(r0, _RPT)], agg_sh.at[pl.ds(r0, _RPT)])
        plsc.subcore_barrier()
        bufs = (rows0, rows1)
        sems = (sem0, sem1)

        def body(j, carry):
            b = j * _NW + wid
            pltpu.sync_copy(src_hbm.at[pl.ds(b * _BLK, _BLK)], sidx)
            pltpu.sync_copy(dst_hbm.at[pl.ds(b * _BLK, _BLK)], didx)
            descs = [None] * _BLK
            descs[0] = pltpu.async_copy(xs_hbm.at[sidx.at[0]], bufs[0], sems[0])
            for k in range(_BLK):
                if k + 1 < _BLK:
                    descs[k + 1] = pltpu.async_copy(
                        xs_hbm.at[sidx.at[k + 1]], bufs[(k + 1) % 2],
                        sems[(k + 1) % 2])
                descs[k].wait()
                pltpu.sync_copy(bufs[k % 2], agg_sh.at[didx.at[k]], add=True)
            return carry

        lax.fori_loop(0, _BPW, body, 0)
        plsc.subcore_barrier()
        pltpu.sync_copy(agg_sh.at[pl.ds(r0, _RPT)],
                        out_hbm.at[cid, pl.ds(r0, _RPT)])

    return agg_k(xs, src2d, dst2d, zeros_nd)


def _emb_body(h_ref, w_ref, b_ref, degp_ref, x_ref, xs_ref, cin_ref, cout_ref):
    x = jnp.dot(h_ref[...], w_ref[...],
                preferred_element_type=jnp.float32) + b_ref[...]
    deg_out = degp_ref[0, 0:_N, 0:1]
    deg_in = degp_ref[1, 0:_N, 0:1]
    c_out = lax.rsqrt(jnp.maximum(deg_out, 1.0))
    c_in = lax.rsqrt(jnp.maximum(deg_in, 1.0))
    x_ref[...] = x
    xs_ref[0:_N] = x * c_out
    xs_ref[_N:_NP] = jnp.zeros((_NP - _N, _D), jnp.float32)
    cin_ref[...] = c_in
    cout_ref[...] = c_out


def _embed(h, W_emb, b_emb, degp):
    return pl.pallas_call(
        _emb_body,
        out_shape=[
            jax.ShapeDtypeStruct((_N, _D), jnp.float32),
            jax.ShapeDtypeStruct((_NP, _D), jnp.float32),
            jax.ShapeDtypeStruct((_N, 1), jnp.float32),
            jax.ShapeDtypeStruct((_N, 1), jnp.float32),
        ],
    )(h, W_emb, b_emb.reshape(1, _D), degp)


def _layer_body(p_ref, cin_ref, cout_ref, w_ref, b_ref, g_ref, bt_ref,
                xin_ref, xout_ref, xsout_ref):
    agg = (p_ref[0, 0:_N] + p_ref[1, 0:_N]) * cin_ref[...]
    y = jnp.dot(agg, w_ref[...],
                preferred_element_type=jnp.float32) + b_ref[...]
    mu = jnp.mean(y, axis=0, keepdims=True)
    yc = y - mu
    var = jnp.mean(yc * yc, axis=0, keepdims=True)
    yn = yc * lax.rsqrt(var + 1e-5) * g_ref[...] + bt_ref[...]
    x_new = xin_ref[...] + jnp.maximum(yn, 0.0)
    xout_ref[...] = x_new
    xsout_ref[0:_N] = x_new * cout_ref[...]
    xsout_ref[_N:_NP] = jnp.zeros((_NP - _N, _D), jnp.float32)


def _layer(parts, c_in, c_out, W, b, g, bt, x_in):
    return pl.pallas_call(
        _layer_body,
        out_shape=[
            jax.ShapeDtypeStruct((_N, _D), jnp.float32),
            jax.ShapeDtypeStruct((_NP, _D), jnp.float32),
        ],
    )(parts, c_in, c_out, W, b.reshape(1, _D), g.reshape(1, _D),
      bt.reshape(1, _D), x_in)


def kernel(h, edge_index, e, W_emb, b_emb, Ws, bs, gammas, betas):
    del e
    pad = _N + (jnp.arange(_EPAD - _E, dtype=jnp.int32) % (_NP - _N))
    src2d = jnp.concatenate([edge_index[0], pad]).reshape(_PCHUNKS, _CHUNK)
    dst2d = jnp.concatenate([edge_index[1], pad]).reshape(_PCHUNKS, _CHUNK)
    zeros_nd = jnp.zeros((_NP, _D), jnp.float32)
    ones_nd = jnp.ones((_CHUNK, _D), jnp.float32)

    degp = _degree_partials(src2d, dst2d, zeros_nd, ones_nd)
    x, xs, c_in, c_out = _embed(h, W_emb, b_emb, degp)
    for l in range(4):
        parts = _aggregate_partials(xs, src2d, dst2d, zeros_nd)
        x, xs = _layer(parts, c_in, c_out, Ws[l], bs[l], gammas[l],
                       betas[l], x)
    return x


# speedup vs baseline: 2.5501x; 1.0370x over previous
import functools

import jax
import jax.numpy as jnp
from jax import lax
from jax.experimental import pallas as pl
from jax.experimental.pallas import tpu as pltpu
from jax.experimental.pallas import tpu_sc as plsc

_N = 10000
_E = 320000
_D = 128
_NC = 2
_NS = 16
_NW = _NC * _NS
_CHUNK = 128
_NCHUNKS = _E // _CHUNK
_CPW = -(-_NCHUNKS // _NW)
_NP = 10240
_BLK = 8
_PCHUNKS = 2560
_EPAD = _PCHUNKS * _CHUNK
_BPW = _PCHUNKS // (_NW * _BLK)
_WCH = _PCHUNKS // _NW
_NBUF = 5
_DCH = _PCHUNKS // _NS
_DBUF = 4
_RPT = _NP // _NS
_DEGW = 16


def _sc_mesh():
    return plsc.VectorSubcoreMesh(
        core_axis_name="c", subcore_axis_name="s",
        num_cores=_NC, num_subcores=_NS)


_DCP = -(-_NCHUNKS // _NS)


def _degree_partials(src, dst, zeros_nd, ones_nd):
    @functools.partial(
        pl.kernel,
        out_type=jax.ShapeDtypeStruct((_NC, _NP, _D), jnp.float32),
        mesh=_sc_mesh(),
        scratch_types=[
            pltpu.VMEM((_DCH, _CHUNK), jnp.int32),
            pltpu.VMEM((_CHUNK, _D), jnp.float32),
            pltpu.VMEM_SHARED((_NP, _D), jnp.float32),
        ] + [pltpu.SemaphoreType.DMA] * _DBUF,
    )
    def deg_k(src_hbm, dst_hbm, z_hbm, o_hbm, out_hbm, idx, ones_v, acc_sh,
              *ssems):
        cid = lax.axis_index("c")
        sid = lax.axis_index("s")
        r0 = sid * _RPT
        pltpu.sync_copy(z_hbm.at[pl.ds(r0, _RPT)], acc_sh.at[pl.ds(r0, _RPT)])
        pltpu.sync_copy(o_hbm, ones_v)
        c0 = sid * _DCH

        @pl.when(cid == 0)
        def _():
            pltpu.sync_copy(src_hbm.at[pl.ds(c0, _DCH)], idx)

        @pl.when(cid == 1)
        def _():
            pltpu.sync_copy(dst_hbm.at[pl.ds(c0, _DCH)], idx)

        plsc.subcore_barrier()

        def body(i, carry):
            base = i * _DBUF
            for p in range(_DBUF):
                @pl.when(i > 0)
                def _(p=p):
                    pltpu.make_async_copy(o_hbm, ones_v, ssems[p]).wait()

                pltpu.async_copy(ones_v, acc_sh.at[idx.at[base + p]],
                                 ssems[p], add=True)
            return carry

        lax.fori_loop(0, _DCH // _DBUF, body, 0)
        for p in range(_DBUF):
            pltpu.make_async_copy(o_hbm, ones_v, ssems[p]).wait()
        plsc.subcore_barrier()
        pltpu.sync_copy(acc_sh.at[pl.ds(r0, _RPT)],
                        out_hbm.at[cid, pl.ds(r0, _RPT)])

    return deg_k(src, dst, zeros_nd, ones_nd)


def _aggregate_partials(xs, src2d, dst2d, zeros_nd):
    @functools.partial(
        pl.kernel,
        out_type=jax.ShapeDtypeStruct((_NC, _NP, _D), jnp.float32),
        mesh=_sc_mesh(),
        scratch_types=[
            pltpu.VMEM((_BLK, _CHUNK), jnp.int32),
            pltpu.VMEM((_BLK, _CHUNK), jnp.int32),
            pltpu.VMEM((_CHUNK, _D), jnp.float32),
            pltpu.VMEM((_CHUNK, _D), jnp.float32),
            pltpu.VMEM_SHARED((_NP, _D), jnp.float32),
        ] + [pltpu.SemaphoreType.DMA] * 4,
    )
    def agg_k(xs_hbm, src_hbm, dst_hbm, z_hbm, out_hbm,
              sidx, didx, rows0, rows1, agg_sh, gsem0, gsem1, ssem0, ssem1):
        cid = lax.axis_index("c")
        sid = lax.axis_index("s")
        wid = cid * _NS + sid
        r0 = sid * _RPT
        pltpu.sync_copy(z_hbm.at[pl.ds(r0, _RPT)], agg_sh.at[pl.ds(r0, _RPT)])
        plsc.subcore_barrier()
        bufs = (rows0, rows1)
        gsems = (gsem0, gsem1)
        ssems = (ssem0, ssem1)

        def blk_body(j, carry):
            b = wid * _BPW + j
            pltpu.sync_copy(src_hbm.at[pl.ds(b * _BLK, _BLK)], sidx)
            pltpu.sync_copy(dst_hbm.at[pl.ds(b * _BLK, _BLK)], didx)
            descs = [None, None]
            for k in range(_BLK):
                p = k % 2
                if k < 2:
                    @pl.when(j > 0)
                    def _(p=p):
                        pltpu.make_async_copy(xs_hbm.at[sidx.at[0]], bufs[p],
                                              ssems[p]).wait()
                else:
                    pltpu.make_async_copy(xs_hbm.at[sidx.at[0]], bufs[p],
                                          ssems[p]).wait()
                descs[p] = pltpu.async_copy(xs_hbm.at[sidx.at[k]], bufs[p],
                                            gsems[p])
                if k >= 1:
                    descs[(k - 1) % 2].wait()
                    pltpu.async_copy(bufs[(k - 1) % 2],
                                     agg_sh.at[didx.at[k - 1]],
                                     ssems[(k - 1) % 2], add=True)
            descs[(_BLK - 1) % 2].wait()
            pltpu.async_copy(bufs[(_BLK - 1) % 2],
                             agg_sh.at[didx.at[_BLK - 1]],
                             ssems[(_BLK - 1) % 2], add=True)
            return carry

        lax.fori_loop(0, _BPW, blk_body, 0)
        for p in range(2):
            pltpu.make_async_copy(xs_hbm.at[sidx.at[0]], bufs[p],
                                  ssems[p]).wait()
        plsc.subcore_barrier()
        pltpu.sync_copy(agg_sh.at[pl.ds(r0, _RPT)],
                        out_hbm.at[cid, pl.ds(r0, _RPT)])

    return agg_k(xs, src2d, dst2d, zeros_nd)


def _matmul_body(h_ref, w_ref, b_ref, x_ref):
    x_ref[...] = jnp.dot(h_ref[...], w_ref[...],
                         preferred_element_type=jnp.float32) + b_ref[...]


def _embed_matmul(h, W_emb, b_emb):
    return pl.pallas_call(
        _matmul_body,
        out_shape=jax.ShapeDtypeStruct((_N, _D), jnp.float32),
    )(h, W_emb, b_emb.reshape(1, _D))


def _scale_body(x_ref, degp_ref, xs_ref, cin_ref, cout_ref):
    deg_out = degp_ref[0, 0:_N, 0:1]
    deg_in = degp_ref[1, 0:_N, 0:1]
    c_out = lax.rsqrt(jnp.maximum(deg_out, 1.0))
    c_in = lax.rsqrt(jnp.maximum(deg_in, 1.0))
    xs_ref[0:_N] = x_ref[...] * c_out
    xs_ref[_N:_NP] = jnp.zeros((_NP - _N, _D), jnp.float32)
    cin_ref[...] = c_in
    cout_ref[...] = c_out


def _scale(x, degp):
    return pl.pallas_call(
        _scale_body,
        out_shape=[
            jax.ShapeDtypeStruct((_NP, _D), jnp.float32),
            jax.ShapeDtypeStruct((_N, 1), jnp.float32),
            jax.ShapeDtypeStruct((_N, 1), jnp.float32),
        ],
    )(x, degp)


def _layer_body(p_ref, cin_ref, cout_ref, w_ref, b_ref, g_ref, bt_ref,
                xin_ref, xout_ref, xsout_ref):
    agg = (p_ref[0, 0:_N] + p_ref[1, 0:_N]) * cin_ref[...]
    y = jnp.dot(agg, w_ref[...],
                preferred_element_type=jnp.float32) + b_ref[...]
    mu = jnp.mean(y, axis=0, keepdims=True)
    yc = y - mu
    var = jnp.mean(yc * yc, axis=0, keepdims=True)
    yn = yc * lax.rsqrt(var + 1e-5) * g_ref[...] + bt_ref[...]
    x_new = xin_ref[...] + jnp.maximum(yn, 0.0)
    xout_ref[...] = x_new
    xsout_ref[0:_N] = x_new * cout_ref[...]
    xsout_ref[_N:_NP] = jnp.zeros((_NP - _N, _D), jnp.float32)


def _layer(parts, c_in, c_out, W, b, g, bt, x_in):
    return pl.pallas_call(
        _layer_body,
        out_shape=[
            jax.ShapeDtypeStruct((_N, _D), jnp.float32),
            jax.ShapeDtypeStruct((_NP, _D), jnp.float32),
        ],
    )(parts, c_in, c_out, W, b.reshape(1, _D), g.reshape(1, _D),
      bt.reshape(1, _D), x_in)


def kernel(h, edge_index, e, W_emb, b_emb, Ws, bs, gammas, betas):
    del e
    pad = _N + (jnp.arange(_EPAD - _E, dtype=jnp.int32) % (_NP - _N))
    src2d = jnp.concatenate([edge_index[0], pad]).reshape(_PCHUNKS, _CHUNK)
    dst2d = jnp.concatenate([edge_index[1], pad]).reshape(_PCHUNKS, _CHUNK)
    zeros_nd = jnp.zeros((_NP, _D), jnp.float32)
    ones_nd = jnp.ones((_CHUNK, _D), jnp.float32)

    degp = _degree_partials(src2d, dst2d, zeros_nd, ones_nd)
    x = _embed_matmul(h, W_emb, b_emb)
    xs, c_in, c_out = _scale(x, degp)
    for l in range(4):
        parts = _aggregate_partials(xs, src2d, dst2d, zeros_nd)
        x, xs = _layer(parts, c_in, c_out, Ws[l], bs[l], gammas[l],
                       betas[l], x)
    return x


# speedup vs baseline: 2.7608x; 1.0826x over previous
import functools

import jax
import jax.numpy as jnp
from jax import lax
from jax.experimental import pallas as pl
from jax.experimental.pallas import tpu as pltpu
from jax.experimental.pallas import tpu_sc as plsc

_N = 10000
_E = 320000
_D = 128
_NC = 2
_NS = 16
_NW = _NC * _NS
_CHUNK = 128
_NCHUNKS = _E // _CHUNK
_CPW = -(-_NCHUNKS // _NW)
_NP = 10240
_BLK = 8
_PCHUNKS = 2560
_EPAD = _PCHUNKS * _CHUNK
_BPW = _PCHUNKS // (_NW * _BLK)
_WCH = _PCHUNKS // _NW
_NBUF = 5
_DCH = _PCHUNKS // _NS
_DBUF = 4
_RPT = _NP // _NS
_DEGW = 16


def _sc_mesh():
    return plsc.VectorSubcoreMesh(
        core_axis_name="c", subcore_axis_name="s",
        num_cores=_NC, num_subcores=_NS)


_DCP = -(-_NCHUNKS // _NS)


def _degree_partials(src, dst, zeros_nd, ones_nd):
    @functools.partial(
        pl.kernel,
        out_type=jax.ShapeDtypeStruct((_NC, _NP, _D), jnp.float32),
        mesh=_sc_mesh(),
        scratch_types=[
            pltpu.VMEM((_DCH, _CHUNK), jnp.int32),
            pltpu.VMEM((_CHUNK, _D), jnp.float32),
            pltpu.VMEM_SHARED((_NP, _D), jnp.float32),
        ] + [pltpu.SemaphoreType.DMA] * _DBUF,
    )
    def deg_k(src_hbm, dst_hbm, z_hbm, o_hbm, out_hbm, idx, ones_v, acc_sh,
              *ssems):
        cid = lax.axis_index("c")
        sid = lax.axis_index("s")
        r0 = sid * _RPT
        pltpu.sync_copy(z_hbm.at[pl.ds(r0, _RPT)], acc_sh.at[pl.ds(r0, _RPT)])
        pltpu.sync_copy(o_hbm, ones_v)
        c0 = sid * _DCH

        @pl.when(cid == 0)
        def _():
            pltpu.sync_copy(src_hbm.at[pl.ds(c0, _DCH)], idx)

        @pl.when(cid == 1)
        def _():
            pltpu.sync_copy(dst_hbm.at[pl.ds(c0, _DCH)], idx)

        plsc.subcore_barrier()

        def body(i, carry):
            base = i * _DBUF
            for p in range(_DBUF):
                @pl.when(i > 0)
                def _(p=p):
                    pltpu.make_async_copy(o_hbm, ones_v, ssems[p]).wait()

                pltpu.async_copy(ones_v, acc_sh.at[idx.at[base + p]],
                                 ssems[p], add=True)
            return carry

        lax.fori_loop(0, _DCH // _DBUF, body, 0)
        for p in range(_DBUF):
            pltpu.make_async_copy(o_hbm, ones_v, ssems[p]).wait()
        plsc.subcore_barrier()
        pltpu.sync_copy(acc_sh.at[pl.ds(r0, _RPT)],
                        out_hbm.at[cid, pl.ds(r0, _RPT)])

    return deg_k(src, dst, zeros_nd, ones_nd)


def _aggregate_partials(xs, ed3, zeros_nd):
    @functools.partial(
        pl.kernel,
        out_type=jax.ShapeDtypeStruct((_NC, _NP, _D), jnp.float32),
        mesh=_sc_mesh(),
        scratch_types=[
            pltpu.VMEM((_BLK, 2, _CHUNK), jnp.int32),
            pltpu.VMEM((_BLK, 2, _CHUNK), jnp.int32),
            pltpu.VMEM((_CHUNK, _D), jnp.float32),
            pltpu.VMEM((_CHUNK, _D), jnp.float32),
            pltpu.VMEM_SHARED((_NP, _D), jnp.float32),
        ] + [pltpu.SemaphoreType.DMA] * 6,
    )
    def agg_k(xs_hbm, ed_hbm, z_hbm, out_hbm,
              idx0, idx1, rows0, rows1, agg_sh,
              gsem0, gsem1, ssem0, ssem1, isem0, isem1):
        cid = lax.axis_index("c")
        sid = lax.axis_index("s")
        wid = cid * _NS + sid
        r0 = sid * _RPT
        b0 = wid * _BPW
        idxs = (idx0, idx1)
        bufs = (rows0, rows1)
        gsems = (gsem0, gsem1)
        ssems = (ssem0, ssem1)
        isems = (isem0, isem1)
        pltpu.async_copy(ed_hbm.at[pl.ds(b0 * _BLK, _BLK)], idx0, isem0)
        pltpu.sync_copy(z_hbm.at[pl.ds(r0, _RPT)], agg_sh.at[pl.ds(r0, _RPT)])
        plsc.subcore_barrier()

        def body(i, carry):
            for half in range(2):
                j = 2 * i + half
                slot = idxs[half]
                nslot = idxs[1 - half]
                pltpu.make_async_copy(ed_hbm.at[pl.ds(b0 * _BLK, _BLK)],
                                      slot, isems[half]).wait()
                descs = [None, None]
                for k in range(_BLK):
                    p = k % 2
                    if k < 2:
                        @pl.when(j > 0)
                        def _(p=p):
                            pltpu.make_async_copy(xs_hbm.at[slot.at[0, 0]],
                                                  bufs[p], ssems[p]).wait()
                    else:
                        pltpu.make_async_copy(xs_hbm.at[slot.at[0, 0]],
                                              bufs[p], ssems[p]).wait()
                    if k == 2:
                        @pl.when(j + 1 < _BPW)
                        def _():
                            pltpu.async_copy(
                                ed_hbm.at[pl.ds((b0 + j + 1) * _BLK, _BLK)],
                                nslot, isems[1 - half])
                    descs[p] = pltpu.async_copy(xs_hbm.at[slot.at[k, 0]],
                                                bufs[p], gsems[p])
                    if k >= 1:
                        descs[(k - 1) % 2].wait()
                        pltpu.async_copy(bufs[(k - 1) % 2],
                                         agg_sh.at[slot.at[k - 1, 1]],
                                         ssems[(k - 1) % 2], add=True)
                descs[(_BLK - 1) % 2].wait()
                pltpu.async_copy(bufs[(_BLK - 1) % 2],
                                 agg_sh.at[slot.at[_BLK - 1, 1]],
                                 ssems[(_BLK - 1) % 2], add=True)
            return carry

        lax.fori_loop(0, _BPW // 2, body, 0)
        for p in range(2):
            pltpu.make_async_copy(xs_hbm.at[idx0.at[0, 0]], bufs[p],
                                  ssems[p]).wait()
        plsc.subcore_barrier()
        pltpu.sync_copy(agg_sh.at[pl.ds(r0, _RPT)],
                        out_hbm.at[cid, pl.ds(r0, _RPT)])

    return agg_k(xs, ed3, zeros_nd)


def _matmul_body(h_ref, w_ref, b_ref, x_ref):
    x_ref[...] = jnp.dot(h_ref[...], w_ref[...],
                         preferred_element_type=jnp.float32) + b_ref[...]


def _embed_matmul(h, W_emb, b_emb):
    return pl.pallas_call(
        _matmul_body,
        out_shape=jax.ShapeDtypeStruct((_N, _D), jnp.float32),
    )(h, W_emb, b_emb.reshape(1, _D))


def _scale_body(x_ref, degp_ref, xs_ref, cin_ref, cout_ref):
    deg_out = degp_ref[0, 0:_N, 0:1]
    deg_in = degp_ref[1, 0:_N, 0:1]
    c_out = lax.rsqrt(jnp.maximum(deg_out, 1.0))
    c_in = lax.rsqrt(jnp.maximum(deg_in, 1.0))
    xs_ref[0:_N] = x_ref[...] * c_out
    xs_ref[_N:_NP] = jnp.zeros((_NP - _N, _D), jnp.float32)
    cin_ref[...] = c_in
    cout_ref[...] = c_out


def _scale(x, degp):
    return pl.pallas_call(
        _scale_body,
        out_shape=[
            jax.ShapeDtypeStruct((_NP, _D), jnp.float32),
            jax.ShapeDtypeStruct((_N, 1), jnp.float32),
            jax.ShapeDtypeStruct((_N, 1), jnp.float32),
        ],
    )(x, degp)


def _layer_body(p_ref, cin_ref, cout_ref, w_ref, b_ref, g_ref, bt_ref,
                xin_ref, xout_ref, xsout_ref):
    agg = (p_ref[0, 0:_N] + p_ref[1, 0:_N]) * cin_ref[...]
    y = jnp.dot(agg, w_ref[...],
                preferred_element_type=jnp.float32) + b_ref[...]
    mu = jnp.mean(y, axis=0, keepdims=True)
    yc = y - mu
    var = jnp.mean(yc * yc, axis=0, keepdims=True)
    yn = yc * lax.rsqrt(var + 1e-5) * g_ref[...] + bt_ref[...]
    x_new = xin_ref[...] + jnp.maximum(yn, 0.0)
    xout_ref[...] = x_new
    xsout_ref[0:_N] = x_new * cout_ref[...]
    xsout_ref[_N:_NP] = jnp.zeros((_NP - _N, _D), jnp.float32)


def _layer(parts, c_in, c_out, W, b, g, bt, x_in):
    return pl.pallas_call(
        _layer_body,
        out_shape=[
            jax.ShapeDtypeStruct((_N, _D), jnp.float32),
            jax.ShapeDtypeStruct((_NP, _D), jnp.float32),
        ],
    )(parts, c_in, c_out, W, b.reshape(1, _D), g.reshape(1, _D),
      bt.reshape(1, _D), x_in)


def kernel(h, edge_index, e, W_emb, b_emb, Ws, bs, gammas, betas):
    del e
    pad = _N + (jnp.arange(_EPAD - _E, dtype=jnp.int32) % (_NP - _N))
    src2d = jnp.concatenate([edge_index[0], pad]).reshape(_PCHUNKS, _CHUNK)
    dst2d = jnp.concatenate([edge_index[1], pad]).reshape(_PCHUNKS, _CHUNK)
    zeros_nd = jnp.zeros((_NP, _D), jnp.float32)
    ones_nd = jnp.ones((_CHUNK, _D), jnp.float32)

    ed3 = jnp.stack([src2d, dst2d], axis=1)
    degp = _degree_partials(src2d, dst2d, zeros_nd, ones_nd)
    x = _embed_matmul(h, W_emb, b_emb)
    xs, c_in, c_out = _scale(x, degp)
    for l in range(4):
        parts = _aggregate_partials(xs, ed3, zeros_nd)
        x, xs = _layer(parts, c_in, c_out, Ws[l], bs[l], gammas[l],
                       betas[l], x)
    return x
